# baseline (speedup 1.0000x reference)
import functools

import jax
import jax.numpy as jnp
from jax.experimental import pallas as pl
from jax.experimental.pallas import tpu as pltpu
from jax.experimental.pallas import tpu_sc as plsc

B = 4096
S = 200
D = 128
H = D // 2
TOKS = B * S
PAIRS = TOKS // 2
GATHER_W = 64
BBH = 16


def _sc_gather_pairs(toki, idx_even, idx_odd):
    mesh = plsc.VectorSubcoreMesh(core_axis_name="c", subcore_axis_name="s")
    num_windows = PAIRS // GATHER_W

    @functools.partial(
        pl.kernel,
        out_type=jax.ShapeDtypeStruct((PAIRS, 2 * H), jnp.int32),
        mesh=mesh,
        compiler_params=pltpu.CompilerParams(use_tc_tiling_on_sc=False),
        scratch_types=[
            pltpu.VMEM((GATHER_W, H), jnp.int32),
            pltpu.VMEM((GATHER_W, H), jnp.int32),
            pltpu.SemaphoreType.DMA,
            pltpu.SemaphoreType.DMA,
        ],
    )
    def gather_kernel(tok_hbm, ie_hbm, io_hbm, out_hbm, scr_e, scr_o,
                      sem_e, sem_o):
        def body(ie_vmem, io_vmem, out_vmem):
            ce = pltpu.make_async_copy(tok_hbm.at[ie_vmem.at[0]], scr_e, sem_e)
            co = pltpu.make_async_copy(tok_hbm.at[io_vmem.at[0]], scr_o, sem_o)
            ce.start()
            co.start()
            ce.wait()
            co.wait()

            @pl.loop(0, GATHER_W)
            def _(r):
                for c in range(H // 16):
                    out_vmem[r, pl.ds(c * 16, 16)] = scr_e[r, pl.ds(c * 16, 16)]
                    out_vmem[r, pl.ds(H + c * 16, 16)] = (
                        scr_o[r, pl.ds(c * 16, 16)])

        pltpu.emit_pipeline(
            body,
            grid=(num_windows,),
            in_specs=[
                pl.BlockSpec((1, GATHER_W), index_map=lambda i: (0, i)),
                pl.BlockSpec((1, GATHER_W), index_map=lambda i: (0, i)),
            ],
            out_specs=[
                pl.BlockSpec((GATHER_W, 2 * H), index_map=lambda i: (i, 0)),
            ],
            core_axis_name=("c", "s"),
            dimension_semantics=(pltpu.PARALLEL,),
        )(ie_hbm, io_hbm, out_hbm)

    return gather_kernel(toki, idx_even.reshape(1, PAIRS),
                         idx_odd.reshape(1, PAIRS))


def _ln_body(g_ref, seg_ref, pos_lo_ref, pos_hi_ref, segd_lo_ref, segd_hi_ref,
             gam_lo_ref, gam_hi_ref, bet_lo_ref, bet_hi_ref, rmat_ref, bd_ref,
             o_ref):
    gi = g_ref[...]
    lo = jax.lax.bitcast_convert_type(
        jax.lax.shift_left(gi, jnp.int32(16)), jnp.float32)
    hi = jax.lax.bitcast_convert_type(
        jnp.bitwise_and(gi, jnp.int32(-65536)), jnp.float32)
    dims = (((2,), (0,)), ((), ()))
    segmat = jax.lax.dot_general(seg_ref[...], rmat_ref[...], dims)
    h_lo = lo + pos_lo_ref[...] + segmat * segd_lo_ref[...]
    h_hi = hi + pos_hi_ref[...] + segmat * segd_hi_ref[...]
    mu = jax.lax.dot_general(h_lo + h_hi, bd_ref[...], dims)
    sq = jax.lax.dot_general(h_lo * h_lo + h_hi * h_hi, bd_ref[...], dims)
    inv = jax.lax.rsqrt(sq - mu * mu + 1e-5)
    o_lo = (h_lo - mu) * inv * gam_lo_ref[...] + bet_lo_ref[...]
    o_hi = (h_hi - mu) * inv * gam_hi_ref[...] + bet_hi_ref[...]
    o_ref[:, 0] = jnp.concatenate(
        [o_lo[..., :H], o_hi[..., :H]], axis=-1)
    o_ref[:, 1] = jnp.concatenate(
        [o_lo[..., H:], o_hi[..., H:]], axis=-1)


def _pairlane(v):
    return jnp.concatenate([v, v]).reshape(1, 1, 2 * H)


def kernel(x, seg, tok_table, pos_table, seg_table, ln_gamma, ln_beta):
    tokbf = tok_table.astype(jnp.bfloat16)
    vocab = tok_table.shape[0]
    packed = jnp.stack([tokbf[:, :H], tokbf[:, H:]], axis=-1)
    toki = jax.lax.bitcast_convert_type(packed, jnp.int32).reshape(vocab, H)

    xr = x.astype(jnp.int32).reshape(B // 2, 2, S)
    idx_even = xr[:, 0, :].reshape(-1)
    idx_odd = xr[:, 1, :].reshape(-1)
    segf2 = seg.astype(jnp.float32).reshape(B // 2, 2, S).transpose(0, 2, 1)

    posP = pos_table[:S] + seg_table[0][None, :]
    pos_lo = jnp.concatenate([posP[:, :H], posP[:, :H]], axis=-1)[None]
    pos_hi = jnp.concatenate([posP[:, H:], posP[:, H:]], axis=-1)[None]
    segd = seg_table[1] - seg_table[0]
    segd_lo = _pairlane(segd[:H])
    segd_hi = _pairlane(segd[H:])
    gam_lo = _pairlane(ln_gamma[:H])
    gam_hi = _pairlane(ln_gamma[H:])
    bet_lo = _pairlane(ln_beta[:H])
    bet_hi = _pairlane(ln_beta[H:])
    rmat = jnp.concatenate(
        [jnp.concatenate([jnp.ones((1, H)), jnp.zeros((1, H))], axis=1),
         jnp.concatenate([jnp.zeros((1, H)), jnp.ones((1, H))], axis=1)],
        axis=0).astype(jnp.float32)
    bd = jnp.kron(jnp.eye(2, dtype=jnp.float32),
                  jnp.full((H, H), 1.0 / D, jnp.float32))

    gathered = _sc_gather_pairs(toki, idx_even, idx_odd)
    gi3 = gathered.reshape(B // 2, S, 2 * H)

    small = [
        pl.BlockSpec((1, S, D), lambda i: (0, 0, 0)),
        pl.BlockSpec((1, S, D), lambda i: (0, 0, 0)),
        pl.BlockSpec((1, 1, D), lambda i: (0, 0, 0)),
        pl.BlockSpec((1, 1, D), lambda i: (0, 0, 0)),
        pl.BlockSpec((1, 1, D), lambda i: (0, 0, 0)),
        pl.BlockSpec((1, 1, D), lambda i: (0, 0, 0)),
        pl.BlockSpec((1, 1, D), lambda i: (0, 0, 0)),
        pl.BlockSpec((1, 1, D), lambda i: (0, 0, 0)),
        pl.BlockSpec((2, D), lambda i: (0, 0)),
        pl.BlockSpec((D, D), lambda i: (0, 0)),
    ]
    out = pl.pallas_call(
        _ln_body,
        grid=((B // 2) // BBH,),
        in_specs=[
            pl.BlockSpec((BBH, S, D), lambda i: (i, 0, 0)),
            pl.BlockSpec((BBH, S, 2), lambda i: (i, 0, 0)),
        ] + small,
        out_specs=pl.BlockSpec((BBH, 2, S, D), lambda i: (i, 0, 0, 0)),
        out_shape=jax.ShapeDtypeStruct((B // 2, 2, S, D), jnp.float32),
    )(gi3, segf2, pos_lo, pos_hi, segd_lo, segd_hi, gam_lo, gam_hi,
      bet_lo, bet_hi, rmat, bd)
    return out.reshape(B, S, D)


# speedup vs baseline: 1.2621x; 1.2621x over previous
import functools

import jax
import jax.numpy as jnp
from jax.experimental import pallas as pl
from jax.experimental.pallas import tpu as pltpu
from jax.experimental.pallas import tpu_sc as plsc

B = 4096
S = 200
D = 128
TOKS = B * S
GATHER_W = 128
BB = 32


def _sc_gather(tok_table, x_flat, n_rows):
    mesh = plsc.VectorSubcoreMesh(core_axis_name="c", subcore_axis_name="s")
    num_windows = n_rows // GATHER_W

    @functools.partial(
        pl.kernel,
        out_type=jax.ShapeDtypeStruct((n_rows, D), jnp.float32),
        mesh=mesh,
    )
    def gather_kernel(tok_hbm, idx_hbm, out_hbm):
        def body(idx_vmem, out_vmem):
            pltpu.sync_copy(tok_hbm.at[idx_vmem.at[0]], out_vmem)

        pltpu.emit_pipeline(
            body,
            grid=(num_windows,),
            in_specs=[pl.BlockSpec((1, GATHER_W), index_map=lambda i: (0, i))],
            out_specs=[pl.BlockSpec((GATHER_W, D), index_map=lambda i: (i, 0))],
            core_axis_name=("c", "s"),
            dimension_semantics=(pltpu.PARALLEL,),
        )(idx_hbm, out_hbm)

    return gather_kernel(tok_table, x_flat.reshape(1, n_rows))


def _ln_body(g_ref, seg_ref, pos_ref, segd_ref, gam_ref, bet_ref, o_ref):
    segb = seg_ref[...]
    h = g_ref[...] + pos_ref[...] + segb * segd_ref[...]
    ones = jnp.full((D, D), 1.0 / D, jnp.float32)
    dims = (((2,), (0,)), ((), ()))
    mu = jax.lax.dot_general(h, ones, dims)
    sq = jax.lax.dot_general(h * h, ones, dims)
    var = sq - mu * mu
    o_ref[...] = (h - mu) * jax.lax.rsqrt(var + 1e-5) * gam_ref[...] + bet_ref[...]


def kernel(x, seg, tok_table, pos_table, seg_table, ln_gamma, ln_beta):
    x_flat = x.reshape(-1).astype(jnp.int32)
    segf = seg.astype(jnp.float32).reshape(B, S, 1)
    pos3 = (pos_table[:S] + seg_table[0][None, :]).reshape(1, S, D)
    segd = (seg_table[1] - seg_table[0]).reshape(1, 1, D)
    gamma = ln_gamma.reshape(1, 1, D)
    beta = ln_beta.reshape(1, 1, D)

    gathered = _sc_gather(tok_table, x_flat, TOKS).reshape(B, S, D)
    return pl.pallas_call(
        _ln_body,
        grid=(B // BB,),
        in_specs=[
            pl.BlockSpec((BB, S, D), lambda i: (i, 0, 0)),
            pl.BlockSpec((BB, S, 1), lambda i: (i, 0, 0)),
            pl.BlockSpec((1, S, D), lambda i: (0, 0, 0)),
            pl.BlockSpec((1, 1, D), lambda i: (0, 0, 0)),
            pl.BlockSpec((1, 1, D), lambda i: (0, 0, 0)),
            pl.BlockSpec((1, 1, D), lambda i: (0, 0, 0)),
        ],
        out_specs=pl.BlockSpec((BB, S, D), lambda i: (i, 0, 0)),
        out_shape=jax.ShapeDtypeStruct((B, S, D), jnp.float32),
    )(gathered, segf, pos3, segd, gamma, beta)
